# baseline (speedup 1.0000x reference)
import jax
import jax.numpy as jnp
import numpy as np
from jax import lax
from jax.experimental import pallas as pl
from jax.experimental.pallas import tpu as pltpu
from jax.experimental.pallas import tpu_sc as plsc

N = 10000
E = 320000
D = 128
H = 64
G = 16
NC, NS = 2, 16
K = 128
NCH = 160
TOT = NS * NCH * K
NSLOT = 4
ACC_R = 10240
RB = 400
NRB = N // RB

_SC_PARAMS = pltpu.CompilerParams(use_tc_tiling_on_sc=False,
                                 needs_layout_passes=False)

_P_OF_Q = np.array([32 * (q // 32)
                    + (2 * (q % 32) if q % 32 < 16 else 2 * (q % 32 - 16) + 1)
                    for q in range(D)])
_INV_PERM = np.empty(D, dtype=np.int32)
_INV_PERM[_P_OF_Q] = np.arange(D, dtype=np.int32)


def _mesh():
    return plsc.VectorSubcoreMesh(core_axis_name="c", subcore_axis_name="s")


def _deg_body(dsts_hbm, zeros_hbm, out_hbm, dst_v, ones_v, acc, sem):
    c = lax.axis_index("c")
    s = lax.axis_index("s")
    pltpu.sync_copy(dsts_hbm.at[s], dst_v)
    for i in range(K // 16):
        ones_v[pl.ds(i * 16, 16)] = jnp.full((16,), 1.0, jnp.float32)
    sl = ACC_R // NS
    pltpu.sync_copy(zeros_hbm.at[pl.ds(s * sl, sl)], acc.at[pl.ds(s * sl, sl)])
    plsc.subcore_barrier()
    half = NCH // NC
    base = c * half
    cnt = half

    def fire(k, carry):
        pltpu.async_copy(ones_v, acc.at[dst_v.at[base + k]], sem, add=True)
        return carry

    lax.fori_loop(0, cnt, fire, 0)

    def drain(k, carry):
        pltpu.make_async_copy(ones_v, acc.at[dst_v.at[0]], sem).wait()
        return carry

    lax.fori_loop(0, cnt, drain, 0)
    plsc.subcore_barrier()
    pltpu.sync_copy(acc.at[pl.ds(s * sl, sl)],
                    out_hbm.at[c, 0, pl.ds(s * sl, sl)])


def _deg_counts(dsts, zeros1d):
    return pl.kernel(
        _deg_body,
        out_type=jax.ShapeDtypeStruct((NC, 1, ACC_R), jnp.float32),
        mesh=_mesh(),
        scratch_types=[
            pltpu.VMEM((NCH, K), jnp.int32),
            pltpu.VMEM((K,), jnp.float32),
            pltpu.VMEM_SHARED((ACC_R,), jnp.float32),
            pltpu.SemaphoreType.DMA,
        ],
        compiler_params=_SC_PARAMS,
    )(dsts, zeros1d)


def _mp_body(tbl_hbm, srcs_hbm, dsts_hbm, zeros_hbm, out_hbm,
             src_v, dst_v, bb, bf, acc, gsem, ssem):
    c = lax.axis_index("c")
    s = lax.axis_index("s")
    pltpu.sync_copy(srcs_hbm.at[c, s], src_v)
    pltpu.sync_copy(dsts_hbm.at[s], dst_v)
    sl = ACC_R // NS
    pltpu.sync_copy(zeros_hbm.at[pl.ds(s * sl, sl)], acc.at[pl.ds(s * sl, sl)])
    plsc.subcore_barrier()

    mask = jnp.full((16,), -65536, jnp.int32)

    def fire_g(j, slot):
        pltpu.async_copy(tbl_hbm.at[src_v.at[j]], bb.at[slot], gsem)

    def drain_g_pair():
        pltpu.make_async_copy(tbl_hbm.at[src_v.at[0]],
                              bb.at[pl.ds(0, 2)], gsem).wait()

    def fire_s(j, slot):
        pltpu.async_copy(bf.at[slot], acc.at[dst_v.at[j]], ssem, add=True)

    def drain_s_pair():
        pltpu.make_async_copy(bf.at[0], acc.at[dst_v.at[0]], ssem).wait()
        pltpu.make_async_copy(bf.at[1], acc.at[dst_v.at[0]], ssem).wait()

    def convert(bslot, fslot):
        def rows(i, carry):
            for u in range(8):
                r = i * 8 + u
                for g in range(2):
                    w = plsc.bitcast(bb[bslot, r, pl.ds(32 * g, 32)],
                                     jnp.int32)
                    bf[fslot, r, pl.ds(32 * g, 16)] = plsc.bitcast(
                        w << 16, jnp.float32)
                    bf[fslot, r, pl.ds(32 * g + 16, 16)] = plsc.bitcast(
                        w & mask, jnp.float32)
            return carry

        lax.fori_loop(0, K // 8, rows, 0)

    fire_g(0, 0)
    fire_g(1, 1)
    npair = NCH // 2

    def ring(p, carry):
        b2 = (p % 2) * 2
        j = 2 * p
        drain_g_pair()

        @pl.when(p >= 1)
        def _():
            drain_s_pair()

        @pl.when(p + 1 < npair)
        def _():
            fire_g(j + 2, (b2 + 2) % NSLOT)
            fire_g(j + 3, (b2 + 3) % NSLOT)

        convert(b2, 0)
        convert(b2 + 1, 1)
        fire_s(j, 0)
        fire_s(j + 1, 1)
        return carry

    lax.fori_loop(0, npair, ring, 0)
    drain_s_pair()
    plsc.subcore_barrier()
    pltpu.sync_copy(acc.at[pl.ds(s * sl, sl)], out_hbm.at[c, pl.ds(s * sl, sl)])


def _msg_pass(tbl, srcs, dsts, zeros2d):
    return pl.kernel(
        _mp_body,
        out_type=jax.ShapeDtypeStruct((NC, ACC_R, H), jnp.float32),
        mesh=_mesh(),
        scratch_types=[
            pltpu.VMEM((NCH, K), jnp.int32),
            pltpu.VMEM((NCH, K), jnp.int32),
            pltpu.VMEM((NSLOT, K, H), jnp.bfloat16),
            pltpu.VMEM((2, K, H), jnp.float32),
            pltpu.VMEM_SHARED((ACC_R, H), jnp.float32),
            pltpu.SemaphoreType.DMA,
            pltpu.SemaphoreType.DMA,
        ],
        compiler_params=_SC_PARAMS,
    )(tbl, srcs, dsts, zeros2d)


def _dinv_of(d0_ref, d1_ref):
    return lax.rsqrt(d0_ref[...] + d1_ref[...] + 1.0)


def _stageB_body(x_ref, w_ref, wc_ref, d0_ref, d1_ref, h_ref, tbl_ref):
    dinv = _dinv_of(d0_ref, d1_ref)
    x = x_ref[...]
    h_ref[...] = jnp.dot(x, w_ref[...],
                         preferred_element_type=jnp.float32) * dinv
    tbl_ref[...] = (jnp.dot(x, wc_ref[...], preferred_element_type=jnp.float32)
                    * dinv).astype(jnp.bfloat16)


def _stageB(x, w, wc, d0, d1):
    return pl.pallas_call(
        _stageB_body,
        grid=(NRB,),
        in_specs=[
            pl.BlockSpec((RB, D), lambda i: (i, 0)),
            pl.BlockSpec((D, D), lambda i: (0, 0)),
            pl.BlockSpec((D, D), lambda i: (0, 0)),
            pl.BlockSpec((RB, 1), lambda i: (i, 0)),
            pl.BlockSpec((RB, 1), lambda i: (i, 0)),
        ],
        out_specs=[
            pl.BlockSpec((RB, D), lambda i: (i, 0)),
            pl.BlockSpec((RB, D), lambda i: (i, 0)),
        ],
        out_shape=[
            jax.ShapeDtypeStruct((N, D), jnp.float32),
            jax.ShapeDtypeStruct((N, D), jnp.bfloat16),
        ],
    )(x, w, wc, d0, d1)


def _stageD_body(acc_ref, h_ref, d0_ref, d1_ref, w_ref, wc_ref, b_ref,
                 h2_ref, tbl_ref):
    dinv = _dinv_of(d0_ref, d1_ref)
    z = jnp.concatenate([acc_ref[0], acc_ref[1]], axis=1) + h_ref[...]
    a = jnp.maximum(dinv * z + b_ref[...][None, :], 0.0)
    h2_ref[...] = jnp.dot(a, w_ref[...],
                          preferred_element_type=jnp.float32) * dinv
    tbl_ref[...] = (jnp.dot(a, wc_ref[...], preferred_element_type=jnp.float32)
                    * dinv).astype(jnp.bfloat16)


def _stageD(acc, h, d0, d1, w, wc, b):
    return pl.pallas_call(
        _stageD_body,
        grid=(NRB,),
        in_specs=[
            pl.BlockSpec((NC, RB, H), lambda i: (0, i, 0)),
            pl.BlockSpec((RB, D), lambda i: (i, 0)),
            pl.BlockSpec((RB, 1), lambda i: (i, 0)),
            pl.BlockSpec((RB, 1), lambda i: (i, 0)),
            pl.BlockSpec((D, D), lambda i: (0, 0)),
            pl.BlockSpec((D, D), lambda i: (0, 0)),
            pl.BlockSpec((D,), lambda i: (0,)),
        ],
        out_specs=[
            pl.BlockSpec((RB, D), lambda i: (i, 0)),
            pl.BlockSpec((RB, D), lambda i: (i, 0)),
        ],
        out_shape=[
            jax.ShapeDtypeStruct((N, D), jnp.float32),
            jax.ShapeDtypeStruct((N, D), jnp.bfloat16),
        ],
    )(acc, h, d0, d1, w, wc, b)


def _stageF_body(acc_ref, h_ref, d0_ref, d1_ref, b2_ref, batch_ref,
                 wm1_ref, bm1_ref, wm2_ref, out_ref, gsum, cnt):
    i = pl.program_id(0)
    dinv = _dinv_of(d0_ref, d1_ref)
    z = jnp.concatenate([acc_ref[0], acc_ref[1]], axis=1) + h_ref[...]
    a = jnp.maximum(dinv * z + b2_ref[...][None, :], 0.0)
    b3 = batch_ref[0]
    oh = (jnp.broadcast_to(b3, (G, RB))
          == lax.broadcasted_iota(jnp.int32, (G, RB), 0)).astype(jnp.float32)

    @pl.when(i == 0)
    def _():
        gsum[...] = jnp.zeros((G, D), jnp.float32)
        cnt[...] = jnp.zeros((G, D), jnp.float32)

    gsum[...] += jnp.dot(oh, a, preferred_element_type=jnp.float32)
    cnt[...] += jnp.broadcast_to(jnp.sum(oh, axis=1, keepdims=True), (G, D))

    @pl.when(i == NRB - 1)
    def _():
        g = gsum[...] / jnp.maximum(cnt[...], 1.0)
        t = jnp.maximum(
            jnp.dot(g, wm1_ref[...], preferred_element_type=jnp.float32)
            + bm1_ref[...][None, :], 0.0)
        out_ref[...] = jnp.dot(t, wm2_ref[...],
                               preferred_element_type=jnp.float32)


def _stageF(acc, h, d0, d1, b2, batch3d, wm1, bm1, wm2):
    return pl.pallas_call(
        _stageF_body,
        grid=(NRB,),
        in_specs=[
            pl.BlockSpec((NC, RB, H), lambda i: (0, i, 0)),
            pl.BlockSpec((RB, D), lambda i: (i, 0)),
            pl.BlockSpec((RB, 1), lambda i: (i, 0)),
            pl.BlockSpec((RB, 1), lambda i: (i, 0)),
            pl.BlockSpec((D,), lambda i: (0,)),
            pl.BlockSpec((1, 1, RB), lambda i: (i, 0, 0)),
            pl.BlockSpec((D, D), lambda i: (0, 0)),
            pl.BlockSpec((D,), lambda i: (0,)),
            pl.BlockSpec((D, 1), lambda i: (0, 0)),
        ],
        out_specs=pl.BlockSpec((G, 1), lambda i: (0, 0)),
        out_shape=jax.ShapeDtypeStruct((G, 1), jnp.float32),
        scratch_shapes=[
            pltpu.VMEM((G, D), jnp.float32),
            pltpu.VMEM((G, D), jnp.float32),
        ],
    )(acc, h, d0, d1, b2, batch3d, wm1, bm1, wm2)


def kernel(x, edge_index, batch, W1, b1, W2, b2, Wm1, bm1, Wm2, bm2):
    src = edge_index[0]
    dst = edge_index[1]
    pad = TOT - E
    srcp = jnp.concatenate([src, jnp.zeros((pad,), jnp.int32)])
    dstp = jnp.concatenate([dst, jnp.full((pad,), N, jnp.int32)])
    srcp = srcp.reshape(NS, NCH, K)
    dsts = dstp.reshape(NS, NCH, K)
    srcs = jnp.stack([2 * srcp, 2 * srcp + 1])
    zeros1d = jnp.zeros((ACC_R,), jnp.float32)
    zeros2d = jnp.zeros((ACC_R, H), jnp.float32)
    batch3d = batch.reshape(NRB, 1, RB)
    inv = jnp.asarray(_INV_PERM)
    W1c = W1[:, inv]
    W2c = W2[:, inv]

    deg = _deg_counts(dsts, zeros1d)
    d0 = deg[0, 0].reshape(ACC_R, 1)
    d1 = deg[1, 0].reshape(ACC_R, 1)

    h1, tbl1 = _stageB(x, W1, W1c, d0, d1)
    acc1 = _msg_pass(tbl1.reshape(2 * N, H), srcs, dsts, zeros2d)
    h2, tbl2 = _stageD(acc1, h1, d0, d1, W2, W2c, b1)
    acc2 = _msg_pass(tbl2.reshape(2 * N, H), srcs, dsts, zeros2d)
    y = _stageF(acc2, h2, d0, d1, b2, batch3d, Wm1, bm1, Wm2)
    return y.reshape(-1) + bm2


# speedup vs baseline: 1.2173x; 1.2173x over previous
import jax
import jax.numpy as jnp
import numpy as np
from jax import lax
from jax.experimental import pallas as pl
from jax.experimental.pallas import tpu as pltpu
from jax.experimental.pallas import tpu_sc as plsc

N = 10000
E = 320000
D = 128
H = 64
G = 16
NC, NS = 2, 16
K = 128
NCH = 159
TOT = NS * NCH * K
NSLOT = 3
ACC_R = 10240
RB = 400
NRB = N // RB

_SC_PARAMS = pltpu.CompilerParams(use_tc_tiling_on_sc=False,
                                 needs_layout_passes=False)

_P_OF_Q = np.array([32 * (q // 32)
                    + (2 * (q % 32) if q % 32 < 16 else 2 * (q % 32 - 16) + 1)
                    for q in range(D)])
_INV_PERM = np.empty(D, dtype=np.int32)
_INV_PERM[_P_OF_Q] = np.arange(D, dtype=np.int32)


def _mesh():
    return plsc.VectorSubcoreMesh(core_axis_name="c", subcore_axis_name="s")


def _deg_body(dsts_hbm, zeros_hbm, out_hbm, dst_v, ones_v, acc, sem):
    c = lax.axis_index("c")
    s = lax.axis_index("s")
    pltpu.sync_copy(dsts_hbm.at[s], dst_v)
    for i in range(K // 16):
        ones_v[pl.ds(i * 16, 16)] = jnp.full((16,), 1.0, jnp.float32)
    sl = ACC_R // NS
    pltpu.sync_copy(zeros_hbm.at[pl.ds(s * sl, sl)], acc.at[pl.ds(s * sl, sl)])
    plsc.subcore_barrier()
    half = (NCH + 1) // NC
    base = c * half
    cnt = half - c

    def fire(k, carry):
        pltpu.async_copy(ones_v, acc.at[dst_v.at[base + k]], sem, add=True)
        return carry

    lax.fori_loop(0, cnt, fire, 0)

    def drain(k, carry):
        pltpu.make_async_copy(ones_v, acc.at[dst_v.at[0]], sem).wait()
        return carry

    lax.fori_loop(0, cnt, drain, 0)
    plsc.subcore_barrier()
    pltpu.sync_copy(acc.at[pl.ds(s * sl, sl)],
                    out_hbm.at[c, 0, pl.ds(s * sl, sl)])


def _deg_counts(dsts, zeros1d):
    return pl.kernel(
        _deg_body,
        out_type=jax.ShapeDtypeStruct((NC, 1, ACC_R), jnp.float32),
        mesh=_mesh(),
        scratch_types=[
            pltpu.VMEM((NCH, K), jnp.int32),
            pltpu.VMEM((K,), jnp.float32),
            pltpu.VMEM_SHARED((ACC_R,), jnp.float32),
            pltpu.SemaphoreType.DMA,
        ],
        compiler_params=_SC_PARAMS,
    )(dsts, zeros1d)


def _mp_body(tbl_hbm, srcs_hbm, dsts_hbm, zeros_hbm, out_hbm,
             src_v, dst_v, bb, bf, acc, gsem, ssem):
    c = lax.axis_index("c")
    s = lax.axis_index("s")
    pltpu.sync_copy(srcs_hbm.at[c, s], src_v)
    pltpu.sync_copy(dsts_hbm.at[s], dst_v)
    sl = ACC_R // NS
    pltpu.sync_copy(zeros_hbm.at[pl.ds(s * sl, sl)], acc.at[pl.ds(s * sl, sl)])
    plsc.subcore_barrier()

    mask = jnp.full((16,), -65536, jnp.int32)

    def fire_g(j, slot):
        pltpu.async_copy(tbl_hbm.at[src_v.at[j]], bb.at[slot], gsem)

    def drain_g(slot):
        pltpu.make_async_copy(tbl_hbm.at[src_v.at[0]], bb.at[slot],
                              gsem).wait()

    def fire_s(j, slot):
        pltpu.async_copy(bf.at[slot], acc.at[dst_v.at[j]], ssem, add=True)

    def drain_s(slot):
        pltpu.make_async_copy(bf.at[slot], acc.at[dst_v.at[0]], ssem).wait()

    def convert(slot):
        def rows(i, carry):
            for u in range(8):
                r = i * 8 + u
                for g in range(2):
                    w = plsc.bitcast(bb[slot, r, pl.ds(32 * g, 32)],
                                     jnp.int32)
                    bf[slot, r, pl.ds(32 * g, 16)] = plsc.bitcast(
                        w << 16, jnp.float32)
                    bf[slot, r, pl.ds(32 * g + 16, 16)] = plsc.bitcast(
                        w & mask, jnp.float32)
            return carry

        lax.fori_loop(0, K // 8, rows, 0)

    fire_g(0, 0)

    def ring(m, carry):
        for b in range(NSLOT):
            j = m * NSLOT + b
            drain_g(b)

            @pl.when(j >= 2)
            def _():
                drain_s((b + 1) % NSLOT)

            @pl.when(j + 1 < NCH)
            def _():
                fire_g(j + 1, (b + 1) % NSLOT)

            convert(b)
            fire_s(j, b)
        return carry

    lax.fori_loop(0, NCH // NSLOT, ring, 0)
    drain_s((NCH - 2) % NSLOT)
    drain_s((NCH - 1) % NSLOT)
    plsc.subcore_barrier()
    pltpu.sync_copy(acc.at[pl.ds(s * sl, sl)], out_hbm.at[c, pl.ds(s * sl, sl)])


def _msg_pass(tbl, srcs, dsts, zeros2d):
    return pl.kernel(
        _mp_body,
        out_type=jax.ShapeDtypeStruct((NC, ACC_R, H), jnp.float32),
        mesh=_mesh(),
        scratch_types=[
            pltpu.VMEM((NCH, K), jnp.int32),
            pltpu.VMEM((NCH, K), jnp.int32),
            pltpu.VMEM((NSLOT, K, H), jnp.bfloat16),
            pltpu.VMEM((NSLOT, K, H), jnp.float32),
            pltpu.VMEM_SHARED((ACC_R, H), jnp.float32),
            pltpu.SemaphoreType.DMA,
            pltpu.SemaphoreType.DMA,
        ],
        compiler_params=_SC_PARAMS,
    )(tbl, srcs, dsts, zeros2d)


def _dinv_of(d0_ref, d1_ref):
    return lax.rsqrt(d0_ref[...] + d1_ref[...] + 1.0)


def _stageB_body(x_ref, w_ref, wc_ref, d0_ref, d1_ref, h_ref, tbl_ref):
    dinv = _dinv_of(d0_ref, d1_ref)
    x = x_ref[...]
    h_ref[...] = jnp.dot(x, w_ref[...],
                         preferred_element_type=jnp.float32) * dinv
    tbl_ref[...] = (jnp.dot(x, wc_ref[...], preferred_element_type=jnp.float32)
                    * dinv).astype(jnp.bfloat16)


def _stageB(x, w, wc, d0, d1):
    return pl.pallas_call(
        _stageB_body,
        grid=(NRB,),
        in_specs=[
            pl.BlockSpec((RB, D), lambda i: (i, 0)),
            pl.BlockSpec((D, D), lambda i: (0, 0)),
            pl.BlockSpec((D, D), lambda i: (0, 0)),
            pl.BlockSpec((RB, 1), lambda i: (i, 0)),
            pl.BlockSpec((RB, 1), lambda i: (i, 0)),
        ],
        out_specs=[
            pl.BlockSpec((RB, D), lambda i: (i, 0)),
            pl.BlockSpec((RB, D), lambda i: (i, 0)),
        ],
        out_shape=[
            jax.ShapeDtypeStruct((N, D), jnp.float32),
            jax.ShapeDtypeStruct((N, D), jnp.bfloat16),
        ],
    )(x, w, wc, d0, d1)


def _stageD_body(acc_ref, h_ref, d0_ref, d1_ref, w_ref, wc_ref, b_ref,
                 h2_ref, tbl_ref):
    dinv = _dinv_of(d0_ref, d1_ref)
    z = jnp.concatenate([acc_ref[0], acc_ref[1]], axis=1) + h_ref[...]
    a = jnp.maximum(dinv * z + b_ref[...][None, :], 0.0)
    h2_ref[...] = jnp.dot(a, w_ref[...],
                          preferred_element_type=jnp.float32) * dinv
    tbl_ref[...] = (jnp.dot(a, wc_ref[...], preferred_element_type=jnp.float32)
                    * dinv).astype(jnp.bfloat16)


def _stageD(acc, h, d0, d1, w, wc, b):
    return pl.pallas_call(
        _stageD_body,
        grid=(NRB,),
        in_specs=[
            pl.BlockSpec((NC, RB, H), lambda i: (0, i, 0)),
            pl.BlockSpec((RB, D), lambda i: (i, 0)),
            pl.BlockSpec((RB, 1), lambda i: (i, 0)),
            pl.BlockSpec((RB, 1), lambda i: (i, 0)),
            pl.BlockSpec((D, D), lambda i: (0, 0)),
            pl.BlockSpec((D, D), lambda i: (0, 0)),
            pl.BlockSpec((D,), lambda i: (0,)),
        ],
        out_specs=[
            pl.BlockSpec((RB, D), lambda i: (i, 0)),
            pl.BlockSpec((RB, D), lambda i: (i, 0)),
        ],
        out_shape=[
            jax.ShapeDtypeStruct((N, D), jnp.float32),
            jax.ShapeDtypeStruct((N, D), jnp.bfloat16),
        ],
    )(acc, h, d0, d1, w, wc, b)


def _stageF_body(acc_ref, h_ref, d0_ref, d1_ref, b2_ref, batch_ref,
                 wm1_ref, bm1_ref, wm2_ref, out_ref, gsum, cnt):
    i = pl.program_id(0)
    dinv = _dinv_of(d0_ref, d1_ref)
    z = jnp.concatenate([acc_ref[0], acc_ref[1]], axis=1) + h_ref[...]
    a = jnp.maximum(dinv * z + b2_ref[...][None, :], 0.0)
    b3 = batch_ref[0]
    oh = (jnp.broadcast_to(b3, (G, RB))
          == lax.broadcasted_iota(jnp.int32, (G, RB), 0)).astype(jnp.float32)

    @pl.when(i == 0)
    def _():
        gsum[...] = jnp.zeros((G, D), jnp.float32)
        cnt[...] = jnp.zeros((G, D), jnp.float32)

    gsum[...] += jnp.dot(oh, a, preferred_element_type=jnp.float32)
    cnt[...] += jnp.broadcast_to(jnp.sum(oh, axis=1, keepdims=True), (G, D))

    @pl.when(i == NRB - 1)
    def _():
        g = gsum[...] / jnp.maximum(cnt[...], 1.0)
        t = jnp.maximum(
            jnp.dot(g, wm1_ref[...], preferred_element_type=jnp.float32)
            + bm1_ref[...][None, :], 0.0)
        out_ref[...] = jnp.dot(t, wm2_ref[...],
                               preferred_element_type=jnp.float32)


def _stageF(acc, h, d0, d1, b2, batch3d, wm1, bm1, wm2):
    return pl.pallas_call(
        _stageF_body,
        grid=(NRB,),
        in_specs=[
            pl.BlockSpec((NC, RB, H), lambda i: (0, i, 0)),
            pl.BlockSpec((RB, D), lambda i: (i, 0)),
            pl.BlockSpec((RB, 1), lambda i: (i, 0)),
            pl.BlockSpec((RB, 1), lambda i: (i, 0)),
            pl.BlockSpec((D,), lambda i: (0,)),
            pl.BlockSpec((1, 1, RB), lambda i: (i, 0, 0)),
            pl.BlockSpec((D, D), lambda i: (0, 0)),
            pl.BlockSpec((D,), lambda i: (0,)),
            pl.BlockSpec((D, 1), lambda i: (0, 0)),
        ],
        out_specs=pl.BlockSpec((G, 1), lambda i: (0, 0)),
        out_shape=jax.ShapeDtypeStruct((G, 1), jnp.float32),
        scratch_shapes=[
            pltpu.VMEM((G, D), jnp.float32),
            pltpu.VMEM((G, D), jnp.float32),
        ],
    )(acc, h, d0, d1, b2, batch3d, wm1, bm1, wm2)


def kernel(x, edge_index, batch, W1, b1, W2, b2, Wm1, bm1, Wm2, bm2):
    src = edge_index[0]
    dst = edge_index[1]
    pad = TOT - E
    srcp = jnp.concatenate([src, jnp.zeros((pad,), jnp.int32)])
    dstp = jnp.concatenate([dst, jnp.full((pad,), N, jnp.int32)])
    srcp = srcp.reshape(NS, NCH, K)
    dsts = dstp.reshape(NS, NCH, K)
    srcs = jnp.stack([2 * srcp, 2 * srcp + 1])
    zeros1d = jnp.zeros((ACC_R,), jnp.float32)
    zeros2d = jnp.zeros((ACC_R, H), jnp.float32)
    batch3d = batch.reshape(NRB, 1, RB)
    inv = jnp.asarray(_INV_PERM)
    W1c = W1[:, inv]
    W2c = W2[:, inv]

    deg = _deg_counts(dsts, zeros1d)
    d0 = deg[0, 0].reshape(ACC_R, 1)
    d1 = deg[1, 0].reshape(ACC_R, 1)

    h1, tbl1 = _stageB(x, W1, W1c, d0, d1)
    acc1 = _msg_pass(tbl1.reshape(2 * N, H), srcs, dsts, zeros2d)
    h2, tbl2 = _stageD(acc1, h1, d0, d1, W2, W2c, b1)
    acc2 = _msg_pass(tbl2.reshape(2 * N, H), srcs, dsts, zeros2d)
    y = _stageF(acc2, h2, d0, d1, b2, batch3d, Wm1, bm1, Wm2)
    return y.reshape(-1) + bm2
